# baseline (speedup 1.0000x reference)
import functools

import jax
import jax.numpy as jnp
from jax import lax
from jax.experimental import pallas as pl
from jax.experimental.pallas import tpu as pltpu
from jax.experimental.pallas import tpu_sc as plsc

NC = 2
NS = 16
NW = NC * NS

CH = 128


def kernel(x, weight):
    R, C = x.shape
    D = weight.shape[1]
    B = R * C
    bpw = B // NW
    nch = bpw // CH
    wpr = C // bpw
    assert bpw % CH == 0 and C % bpw == 0

    mesh = plsc.VectorSubcoreMesh(core_axis_name="c", subcore_axis_name="s")
    NBUF = 7

    @functools.partial(
        pl.kernel,
        mesh=mesh,
        out_type=jax.ShapeDtypeStruct((R, C, D), jnp.float32),
        scratch_types=(
            [pltpu.VMEM((bpw,), jnp.int32)]
            + [pltpu.VMEM((CH, D), jnp.float32) for _ in range(NBUF)]
            + [pltpu.SemaphoreType.DMA for _ in range(2 * NBUF + 1)]
        ),
    )
    def k(idx_hbm, table_hbm, out_hbm, idx_v, *rest):
        bufs = rest[:NBUF]
        gsem = rest[NBUF:2 * NBUF]
        ssem = rest[2 * NBUF:3 * NBUF]
        isem = rest[3 * NBUF]
        wid = lax.axis_index("s") * NC + lax.axis_index("c")
        row = wid // wpr
        col = (wid % wpr) * bpw
        pltpu.sync_copy(idx_hbm.at[row, pl.ds(col, CH)],
                        idx_v.at[pl.ds(0, CH)])
        rest_idx = pltpu.async_copy(
            idx_hbm.at[row, pl.ds(col + CH, bpw - CH)],
            idx_v.at[pl.ds(CH, bpw - CH)], isem)

        gathers = [None] * nch
        stores = [None] * nch
        for j in range(min(NBUF - 1, nch)):
            if j == 1:
                rest_idx.wait()
            gathers[j] = pltpu.async_copy(
                table_hbm.at[idx_v.at[pl.ds(j * CH, CH)]], bufs[j % NBUF],
                gsem[j % NBUF])
        for j in range(nch):
            gathers[j].wait()
            stores[j] = pltpu.async_copy(
                bufs[j % NBUF],
                out_hbm.at[row, pl.ds(col + j * CH, CH)],
                ssem[j % NBUF])
            nxt = j + NBUF - 1
            if nxt < nch:
                prev = nxt - NBUF
                if prev >= 0:
                    stores[prev].wait()
                    stores[prev] = None
                gathers[nxt] = pltpu.async_copy(
                    table_hbm.at[idx_v.at[pl.ds(nxt * CH, CH)]],
                    bufs[nxt % NBUF], gsem[nxt % NBUF])
        for st in stores:
            if st is not None:
                st.wait()

    return k(x, weight)


# speedup vs baseline: 1.0163x; 1.0163x over previous
import functools

import jax
import jax.numpy as jnp
from jax import lax
from jax.experimental import pallas as pl
from jax.experimental.pallas import tpu as pltpu
from jax.experimental.pallas import tpu_sc as plsc

NC = 2
NS = 16
NW = NC * NS

CH = 128


def kernel(x, weight):
    R, C = x.shape
    D = weight.shape[1]
    B = R * C
    bpw = B // NW
    nch = bpw // CH
    wpr = C // bpw
    assert bpw % CH == 0 and C % bpw == 0

    mesh = plsc.VectorSubcoreMesh(core_axis_name="c", subcore_axis_name="s")
    NBUF = 7

    @functools.partial(
        pl.kernel,
        mesh=mesh,
        out_type=jax.ShapeDtypeStruct((R, C, D), jnp.float32),
        scratch_types=[
            pltpu.VMEM((bpw,), jnp.int32),
            pltpu.VMEM((NBUF, CH, D), jnp.float32),
            pltpu.SemaphoreType.DMA((NBUF,)),
            pltpu.SemaphoreType.DMA((NBUF,)),
        ],
    )
    def k(idx_hbm, table_hbm, out_hbm, idx_v, buf_v, gsems, ssems):
        bufs = [buf_v.at[b] for b in range(NBUF)]
        gsem = [gsems.at[b] for b in range(NBUF)]
        ssem = [ssems.at[b] for b in range(NBUF)]
        wid = lax.axis_index("s") * NC + lax.axis_index("c")
        row = wid // wpr
        col = (wid % wpr) * bpw
        pltpu.sync_copy(idx_hbm.at[row, pl.ds(col, bpw)], idx_v)

        gathers = [None] * nch
        stores = [None] * nch
        for j in range(min(NBUF - 1, nch)):
            gathers[j] = pltpu.async_copy(
                table_hbm.at[idx_v.at[pl.ds(j * CH, CH)]], bufs[j % NBUF],
                gsem[j % NBUF])
        for j in range(nch):
            gathers[j].wait()
            stores[j] = pltpu.async_copy(
                bufs[j % NBUF],
                out_hbm.at[row, pl.ds(col + j * CH, CH)],
                ssem[j % NBUF])
            nxt = j + NBUF - 1
            if nxt < nch:
                prev = nxt - NBUF
                if prev >= 0:
                    stores[prev].wait()
                    stores[prev] = None
                gathers[nxt] = pltpu.async_copy(
                    table_hbm.at[idx_v.at[pl.ds(nxt * CH, CH)]],
                    bufs[nxt % NBUF], gsem[nxt % NBUF])
        for st in stores:
            if st is not None:
                st.wait()

    return k(x, weight)


# speedup vs baseline: 1.0266x; 1.0101x over previous
import functools

import jax
import jax.numpy as jnp
from jax import lax
from jax.experimental import pallas as pl
from jax.experimental.pallas import tpu as pltpu
from jax.experimental.pallas import tpu_sc as plsc

NC = 2
NS = 16
NW = NC * NS

CH = 128


def kernel(x, weight):
    R, C = x.shape
    D = weight.shape[1]
    B = R * C
    bpw = B // NW
    nch = bpw // CH
    wpr = C // bpw
    assert bpw % CH == 0 and C % bpw == 0

    mesh = plsc.VectorSubcoreMesh(core_axis_name="c", subcore_axis_name="s")
    NBUF = 6

    @functools.partial(
        pl.kernel,
        mesh=mesh,
        out_type=jax.ShapeDtypeStruct((R, C, D), jnp.float32),
        scratch_types=[
            pltpu.VMEM((bpw,), jnp.int32),
            pltpu.VMEM((NBUF, CH, D), jnp.float32),
            pltpu.SemaphoreType.DMA((NBUF,)),
            pltpu.SemaphoreType.DMA((NBUF,)),
        ],
    )
    def k(idx_hbm, table_hbm, out_hbm, idx_v, buf_v, gsems, ssems):
        bufs = [buf_v.at[b] for b in range(NBUF)]
        gsem = [gsems.at[b] for b in range(NBUF)]
        ssem = [ssems.at[b] for b in range(NBUF)]
        wid = lax.axis_index("s") * NC + lax.axis_index("c")
        row = wid // wpr
        col = (wid % wpr) * bpw
        pltpu.sync_copy(idx_hbm.at[row, pl.ds(col, bpw)], idx_v)

        gathers = [None] * nch
        stores = [None] * nch
        for j in range(min(NBUF - 1, nch)):
            gathers[j] = pltpu.async_copy(
                table_hbm.at[idx_v.at[pl.ds(j * CH, CH)]], bufs[j % NBUF],
                gsem[j % NBUF])
        for j in range(nch):
            gathers[j].wait()
            stores[j] = pltpu.async_copy(
                bufs[j % NBUF],
                out_hbm.at[row, pl.ds(col + j * CH, CH)],
                ssem[j % NBUF])
            nxt = j + NBUF - 1
            if nxt < nch:
                prev = nxt - NBUF
                if prev >= 0:
                    stores[prev].wait()
                    stores[prev] = None
                gathers[nxt] = pltpu.async_copy(
                    table_hbm.at[idx_v.at[pl.ds(nxt * CH, CH)]],
                    bufs[nxt % NBUF], gsem[nxt % NBUF])
        for st in stores:
            if st is not None:
                st.wait()

    return k(x, weight)
